# Initial kernel scaffold; baseline (speedup 1.0000x reference)
import jax
import jax.numpy as jnp
from jax.experimental import pallas as pl


def kernel(x, edge_index, W1, b1, W2, b2, W3, b3):
    raise NotImplementedError("write your pallas kernel here")


# speedup vs baseline: 7.6004x; 7.6004x over previous
import functools

import jax
import jax.numpy as jnp
from jax import lax
from jax.experimental import pallas as pl
from jax.experimental.pallas import tpu as pltpu
from jax.experimental.pallas import tpu_sc as plsc

_N = 100000
_E = 1600000
_NP = 100352
_NP2 = 2 * _NP
_K = 80
_RT = _NP // 16
_RC = _RT // 8
_B = 1024
_GRID = _NP // _B

_MESH = plsc.VectorSubcoreMesh(core_axis_name="c", subcore_axis_name="s")


@functools.partial(
    pl.kernel,
    out_type=jax.ShapeDtypeStruct((2, _NP), jnp.float32),
    mesh=_MESH,
    compiler_params=pltpu.CompilerParams(use_tc_tiling_on_sc=False),
    scratch_types=[
        pltpu.VMEM_SHARED((_NP,), jnp.float32),
        pltpu.VMEM((_RT,), jnp.float32),
        pltpu.VMEM((_K,), jnp.float32),
        pltpu.VMEM((1, _K), jnp.int32),
    ],
)
def _deg_kernel(dst_h, out, deg_sh, zbuf, ones, didx):
    c = lax.axis_index("c")
    t = lax.axis_index("s")

    def zf(i, carry):
        zbuf[pl.ds(i * 16, 16)] = jnp.zeros((16,), jnp.float32)
        return carry

    lax.fori_loop(0, _RT // 16, zf, 0)

    def of(i, carry):
        ones[pl.ds(i * 16, 16)] = jnp.ones((16,), jnp.float32)
        return carry

    lax.fori_loop(0, _K // 16, of, 0)
    pltpu.sync_copy(zbuf, deg_sh.at[pl.ds(t * _RT, _RT)])
    plsc.subcore_barrier()

    ebase = (c * 16 + t) * (_E // 32)

    def blk(j, carry):
        pltpu.sync_copy(dst_h.at[pl.ds(ebase + j * _K, _K)], didx.at[0])
        pltpu.sync_copy(ones, deg_sh.at[didx.at[0]], add=True)
        return carry

    lax.fori_loop(0, (_E // 32) // _K, blk, 0)
    plsc.subcore_barrier()
    pltpu.sync_copy(deg_sh.at[pl.ds(t * _RT, _RT)], zbuf)
    pltpu.sync_copy(zbuf, out.at[c, pl.ds(t * _RT, _RT)])


@functools.partial(
    pl.kernel,
    out_type=jax.ShapeDtypeStruct((_NP2, 16), jnp.float32),
    mesh=_MESH,
    compiler_params=pltpu.CompilerParams(use_tc_tiling_on_sc=False),
    scratch_types=[
        pltpu.VMEM_SHARED((_NP, 16), jnp.float32),
        pltpu.VMEM((_RC, 16), jnp.float32),
        pltpu.VMEM((_K, 16), jnp.float32),
        pltpu.VMEM((_K,), jnp.int32),
        pltpu.VMEM((_K,), jnp.int32),
        pltpu.VMEM((1, _K), jnp.int32),
    ],
)
def _conv16_kernel(g2, src_h, dst_h, out, acc_sh, stage, rows, sidx, gidx, didx):
    c = lax.axis_index("c")
    t = lax.axis_index("s")
    base_r = t * _RT
    coff = c * _NP

    def ib(i, carry):
        r0 = base_r + i * _RC
        pltpu.sync_copy(g2.at[pl.ds(coff + r0, _RC)], stage)
        pltpu.sync_copy(stage, acc_sh.at[pl.ds(r0, _RC)])
        return carry

    lax.fori_loop(0, _RT // _RC, ib, 0)
    plsc.subcore_barrier()

    ebase = t * (_E // 16)

    def blk(j, carry):
        eoff = ebase + j * _K
        pltpu.sync_copy(src_h.at[pl.ds(eoff, _K)], sidx)
        pltpu.sync_copy(dst_h.at[pl.ds(eoff, _K)], didx.at[0])
        for i in range(_K // 16):
            gidx[pl.ds(i * 16, 16)] = sidx[pl.ds(i * 16, 16)] + coff
        pltpu.sync_copy(g2.at[gidx], rows)
        pltpu.sync_copy(rows, acc_sh.at[didx.at[0]], add=True)
        return carry

    lax.fori_loop(0, (_E // 16) // _K, blk, 0)
    plsc.subcore_barrier()

    def wb(i, carry):
        r0 = base_r + i * _RC
        pltpu.sync_copy(acc_sh.at[pl.ds(r0, _RC)], stage)
        pltpu.sync_copy(stage, out.at[pl.ds(coff + r0, _RC)])
        return carry

    lax.fori_loop(0, _RT // _RC, wb, 0)


@functools.partial(
    pl.kernel,
    out_type=jax.ShapeDtypeStruct((2, _NP, 8), jnp.float32),
    mesh=_MESH,
    compiler_params=pltpu.CompilerParams(use_tc_tiling_on_sc=False),
    scratch_types=[
        pltpu.VMEM_SHARED((_NP, 8), jnp.float32),
        pltpu.VMEM((_RC, 8), jnp.float32),
        pltpu.VMEM((_K, 8), jnp.float32),
        pltpu.VMEM((_K,), jnp.int32),
        pltpu.VMEM((1, _K), jnp.int32),
    ],
)
def _conv8_kernel(g3, zeros_hbm, src_h, dst_h, out, acc_sh, stage, rows, sidx, didx):
    c = lax.axis_index("c")
    t = lax.axis_index("s")
    base_r = t * _RT

    @pl.when(c == 0)
    def _():
        def ib(i, carry):
            r0 = base_r + i * _RC
            pltpu.sync_copy(g3.at[pl.ds(r0, _RC)], stage)
            pltpu.sync_copy(stage, acc_sh.at[pl.ds(r0, _RC)])
            return carry

        lax.fori_loop(0, _RT // _RC, ib, 0)

    @pl.when(c == 1)
    def _():
        def ib(i, carry):
            r0 = base_r + i * _RC
            pltpu.sync_copy(zeros_hbm.at[pl.ds(r0, _RC)], stage)
            pltpu.sync_copy(stage, acc_sh.at[pl.ds(r0, _RC)])
            return carry

        lax.fori_loop(0, _RT // _RC, ib, 0)

    plsc.subcore_barrier()

    ebase = (c * 16 + t) * (_E // 32)

    def blk(j, carry):
        eoff = ebase + j * _K
        pltpu.sync_copy(src_h.at[pl.ds(eoff, _K)], sidx)
        pltpu.sync_copy(dst_h.at[pl.ds(eoff, _K)], didx.at[0])
        pltpu.sync_copy(g3.at[sidx], rows)
        pltpu.sync_copy(rows, acc_sh.at[didx.at[0]], add=True)
        return carry

    lax.fori_loop(0, (_E // 32) // _K, blk, 0)
    plsc.subcore_barrier()

    def wb(i, carry):
        r0 = base_r + i * _RC
        pltpu.sync_copy(acc_sh.at[pl.ds(r0, _RC)], stage)
        pltpu.sync_copy(stage, out.at[c, pl.ds(r0, _RC)])
        return carry

    lax.fori_loop(0, _RT // _RC, wb, 0)


def _s_of(deg_ref):
    deg = deg_ref[0] + deg_ref[1] + 1.0
    return lax.rsqrt(deg)


def _tc1_body(deg_ref, x_ref, w_ref, out_ref):
    s = _s_of(deg_ref)
    x = x_ref[...]
    w = w_ref[...]
    g = (x[:, 0:1] * w[0:1, :] + x[:, 1:2] * w[1:2, :] + x[:, 2:3] * w[2:3, :])
    g = s * g
    out_ref[0] = g[:, :16]
    out_ref[1] = g[:, 16:]


def _tcmid_body(deg_ref, acc_ref, b_ref, w_ref, out_ref):
    s = _s_of(deg_ref)
    a = jnp.concatenate([acc_ref[0], acc_ref[1]], axis=-1)
    h = jnp.maximum(s * a + b_ref[...], 0.0)
    g = s * jnp.dot(h, w_ref[...], preferred_element_type=jnp.float32)
    out_ref[0] = g[:, :16]
    out_ref[1] = g[:, 16:]


def _tcpre3_body(deg_ref, acc_ref, b_ref, w_ref, out_ref):
    s = _s_of(deg_ref)
    a = jnp.concatenate([acc_ref[0], acc_ref[1]], axis=-1)
    h = jnp.maximum(s * a + b_ref[...], 0.0)
    out_ref[...] = s * jnp.dot(h, w_ref[...], preferred_element_type=jnp.float32)


def _tcfin_body(deg_ref, acc_ref, b_ref, out_ref):
    s = _s_of(deg_ref)
    out_ref[...] = s * (acc_ref[0] + acc_ref[1]) + b_ref[...]


_DEG_SPEC = pl.BlockSpec((2, _B, 1), lambda i: (0, i, 0))
_ACC_SPEC = pl.BlockSpec((2, _B, 16), lambda i: (0, i, 0))


def _tc1(deg3, x_p, W1):
    return pl.pallas_call(
        _tc1_body,
        grid=(_GRID,),
        in_specs=[
            _DEG_SPEC,
            pl.BlockSpec((_B, 3), lambda i: (i, 0)),
            pl.BlockSpec((3, 32), lambda i: (0, 0)),
        ],
        out_specs=_ACC_SPEC,
        out_shape=jax.ShapeDtypeStruct((2, _NP, 16), jnp.float32),
    )(deg3, x_p, W1)


def _tcmid(deg3, acc, b, W):
    return pl.pallas_call(
        _tcmid_body,
        grid=(_GRID,),
        in_specs=[
            _DEG_SPEC,
            _ACC_SPEC,
            pl.BlockSpec((1, 32), lambda i: (0, 0)),
            pl.BlockSpec((32, 32), lambda i: (0, 0)),
        ],
        out_specs=_ACC_SPEC,
        out_shape=jax.ShapeDtypeStruct((2, _NP, 16), jnp.float32),
    )(deg3, acc, b, W)


def _tcpre3(deg3, acc, b, W):
    return pl.pallas_call(
        _tcpre3_body,
        grid=(_GRID,),
        in_specs=[
            _DEG_SPEC,
            _ACC_SPEC,
            pl.BlockSpec((1, 32), lambda i: (0, 0)),
            pl.BlockSpec((32, 8), lambda i: (0, 0)),
        ],
        out_specs=pl.BlockSpec((_B, 8), lambda i: (i, 0)),
        out_shape=jax.ShapeDtypeStruct((_NP, 8), jnp.float32),
    )(deg3, acc, b, W)


def _tcfin(deg3, acc, b):
    return pl.pallas_call(
        _tcfin_body,
        grid=(_GRID,),
        in_specs=[
            _DEG_SPEC,
            pl.BlockSpec((2, _B, 8), lambda i: (0, i, 0)),
            pl.BlockSpec((1, 8), lambda i: (0, 0)),
        ],
        out_specs=pl.BlockSpec((_B, 8), lambda i: (i, 0)),
        out_shape=jax.ShapeDtypeStruct((_NP, 8), jnp.float32),
    )(deg3, acc, b)


def kernel(x, edge_index, W1, b1, W2, b2, W3, b3):
    x_p = jnp.pad(x, ((0, _NP - _N), (0, 0)))
    src_h = edge_index[0]
    dst_h = edge_index[1]

    deg3 = _deg_kernel(dst_h).reshape(2, _NP, 1)
    g1 = _tc1(deg3, x_p, W1)
    acc1 = _conv16_kernel(g1.reshape(_NP2, 16), src_h, dst_h).reshape(2, _NP, 16)
    g2 = _tcmid(deg3, acc1, b1.reshape(1, 32), W2)
    acc2 = _conv16_kernel(g2.reshape(_NP2, 16), src_h, dst_h).reshape(2, _NP, 16)
    g3 = _tcpre3(deg3, acc2, b2.reshape(1, 32), W3)
    acc3 = _conv8_kernel(g3, jnp.zeros((_NP, 8), jnp.float32), src_h, dst_h)
    out = _tcfin(deg3, acc3, b3.reshape(1, 8))
    return out[:_N]
